# Initial kernel scaffold; baseline (speedup 1.0000x reference)
import jax
import jax.numpy as jnp
from jax.experimental import pallas as pl


def kernel(x, edge_index, W1, b1, W2, b2):
    raise NotImplementedError("write your pallas kernel here")


# speedup vs baseline: 1.8565x; 1.8565x over previous
import functools

import jax
import jax.numpy as jnp
from jax import lax
from jax.experimental import pallas as pl
from jax.experimental.pallas import tpu as pltpu
from jax.experimental.pallas import tpu_sc as plsc

N = 10000
NP = 10240
D = 256
E = 160000
HALF = 128
NSUB = 16
CH = 128
GRP = 8
NGRP = 10
EPT = CH * GRP * NGRP
EPAD = EPT * NSUB
STRIPE = 640
ZROWS = 32
TRASH = 10000
ROWS = 640


def _tc_layer1(x, m2, W1):
    def body(x_ref, m_ref, w_ref, o_ref):
        v = pl.program_id(0)
        xb = x_ref[...]
        mb = jnp.where(v == 0, 1.0, m_ref[0])
        res = jnp.dot(xb * mb, w_ref[...], preferred_element_type=jnp.float32)
        o_ref[0, 0] = res[:, :HALF]
        o_ref[1, 0] = res[:, HALF:]

    return pl.pallas_call(
        body,
        grid=(3, NP // ROWS),
        in_specs=[
            pl.BlockSpec((ROWS, D), lambda v, i: (i, 0)),
            pl.BlockSpec((1, ROWS, D), lambda v, i: (jnp.maximum(v - 1, 0), i, 0)),
            pl.BlockSpec((D, D), lambda v, i: (0, 0)),
        ],
        out_specs=pl.BlockSpec((2, 1, ROWS, HALF), lambda v, i: (0, v, i, 0)),
        out_shape=jax.ShapeDtypeStruct((2, 3, NP, HALF), jnp.float32),
    )(x, m2, W1)


def _tc_layer2(a1, W2):
    def body(lo_ref, hi_ref, w_ref, o_ref):
        lo = jnp.maximum(lo_ref[0, 0], 0.0)
        hi = jnp.maximum(hi_ref[0, 0], 0.0)
        w = w_ref[...]
        res = jnp.dot(lo, w[:HALF, :], preferred_element_type=jnp.float32)
        res += jnp.dot(hi, w[HALF:, :], preferred_element_type=jnp.float32)
        o_ref[0, 0] = res[:, :HALF]
        o_ref[1, 0] = res[:, HALF:]

    return pl.pallas_call(
        body,
        grid=(3, NP // ROWS),
        in_specs=[
            pl.BlockSpec((1, 1, ROWS, HALF), lambda v, i: (0, v, i, 0)),
            pl.BlockSpec((1, 1, ROWS, HALF), lambda v, i: (1, v, i, 0)),
            pl.BlockSpec((D, D), lambda v, i: (0, 0)),
        ],
        out_specs=pl.BlockSpec((2, 1, ROWS, HALF), lambda v, i: (0, v, i, 0)),
        out_shape=jax.ShapeDtypeStruct((2, 3, NP, HALF), jnp.float32),
    )(a1, a1, W2)


def _sc_aggregate(h_flat, src_t, dst_t, keep_t, bias2):
    mesh = plsc.VectorSubcoreMesh(core_axis_name="c", subcore_axis_name="s")

    @functools.partial(
        pl.kernel,
        mesh=mesh,
        out_type=jax.ShapeDtypeStruct((2, 3, NP, HALF), jnp.float32),
        scratch_types=[
            pltpu.VMEM((GRP, CH), jnp.int32),
            pltpu.VMEM((GRP, CH), jnp.int32),
            pltpu.VMEM((GRP, CH), jnp.int32),
            pltpu.VMEM((GRP, CH), jnp.int32),
            pltpu.VMEM((GRP, CH), jnp.int32),
            pltpu.VMEM((2, CH, HALF), jnp.float32),
            pltpu.VMEM((ZROWS, HALF), jnp.float32),
            pltpu.VMEM((HALF,), jnp.float32),
            pltpu.VMEM_SHARED((NP, HALF), jnp.float32),
            pltpu.SemaphoreType.DMA,
            pltpu.SemaphoreType.DMA,
        ],
    )
    def k(h_hbm, src_hbm, dst_hbm, keep_hbm, bias_hbm, out_hbm,
          sstage, dstage, kstage, gidxg, deffg, rows, zbuf, bvm, acc,
          sem0, sem1):
        c = lax.axis_index("c")
        s = lax.axis_index("s")
        pltpu.sync_copy(bias_hbm.at[c], bvm)
        bregs = [bvm[pl.ds(j * 16, 16)] for j in range(8)]

        def fillrow(i, carry):
            for j in range(8):
                zbuf[i, pl.ds(j * 16, 16)] = bregs[j]
            return carry

        lax.fori_loop(0, ZROWS, fillrow, 0)

        base = s * STRIPE
        for v in range(3):
            goff = (c * 3 + v) * NP

            for t in range(STRIPE // ZROWS):
                pltpu.sync_copy(zbuf, acc.at[pl.ds(base + t * ZROWS, ZROWS)])

            plsc.subcore_barrier()

            def group(g, carry):
                pltpu.sync_copy(src_hbm.at[s, g], sstage)
                pltpu.sync_copy(dst_hbm.at[s, g], dstage)
                pltpu.sync_copy(keep_hbm.at[v, s, g], kstage)

                def cmp(r, cc):
                    for j in range(8):
                        sl = pl.ds(j * 16, 16)
                        gidxg[r, sl] = sstage[r, sl] + goff
                        deffg[r, sl] = jnp.where(kstage[r, sl] != 0,
                                                 dstage[r, sl], TRASH)
                    return cc

                lax.fori_loop(0, GRP, cmp, 0)

                def chunk(ch, cc):
                    pltpu.async_copy(h_hbm.at[gidxg.at[ch]], rows.at[0],
                                     sem0).wait()
                    pltpu.sync_copy(rows.at[0], acc.at[deffg.at[ch]],
                                    add=True)
                    return cc

                lax.fori_loop(0, GRP, chunk, 0)
                return carry

            lax.fori_loop(0, NGRP, group, 0)

            plsc.subcore_barrier()

            pltpu.sync_copy(acc.at[pl.ds(base, STRIPE)],
                            out_hbm.at[c, v, pl.ds(base, STRIPE)])

    return k(h_flat, src_t, dst_t, keep_t, bias2)


def kernel(x, edge_index, W1, b1, W2, b2):
    src = edge_index[0]
    dst = edge_index[1]

    base = jax.random.key(42)
    kf1, kf2, ke1, ke2 = jax.random.split(base, 4)
    m1 = jax.random.bernoulli(kf1, 0.6, x.shape).astype(jnp.float32)
    m2 = jax.random.bernoulli(kf2, 0.6, x.shape).astype(jnp.float32)
    ew1 = jax.random.bernoulli(ke1, 0.6, (E,))
    ew2 = jax.random.bernoulli(ke2, 0.6, (E,))

    pad = EPAD - E
    src_t = jnp.pad(src, (0, pad)).reshape(NSUB, NGRP, GRP, CH)
    dst_t = jnp.pad(dst, (0, pad)).reshape(NSUB, NGRP, GRP, CH)
    keep3 = jnp.stack([
        jnp.ones((E,), jnp.int32),
        ew1.astype(jnp.int32),
        ew2.astype(jnp.int32),
    ])
    keep_t = jnp.pad(keep3, ((0, 0), (0, pad))).reshape(3, NSUB, NGRP, GRP, CH)

    x_p = jnp.pad(x, ((0, NP - N), (0, 0)))
    masks = jnp.pad(jnp.stack([m1, m2]), ((0, 0), (0, NP - N), (0, 0)))
    h4 = _tc_layer1(x_p, masks, W1)
    a1 = _sc_aggregate(h4.reshape(6 * NP, HALF), src_t, dst_t, keep_t,
                       b1.reshape(2, HALF))
    h24 = _tc_layer2(a1, W2)
    a2 = _sc_aggregate(h24.reshape(6 * NP, HALF), src_t, dst_t, keep_t,
                       b2.reshape(2, HALF))

    z = jnp.concatenate([a2[0, 0, :N], a2[1, 0, :N]], axis=1)
    z1 = jnp.concatenate([a2[0, 1, :N], a2[1, 1, :N]], axis=1)
    z2 = jnp.concatenate([a2[0, 2, :N], a2[1, 2, :N]], axis=1)
    return (z, z1, z2)


# speedup vs baseline: 2.0428x; 1.1003x over previous
import functools

import jax
import jax.numpy as jnp
from jax import lax
from jax.experimental import pallas as pl
from jax.experimental.pallas import tpu as pltpu
from jax.experimental.pallas import tpu_sc as plsc

N = 10000
NP = 10240
D = 256
E = 160000
HALF = 128
NSUB = 16
CH = 128
GRP = 8
NGRP = 10
EPT = CH * GRP * NGRP
EPAD = EPT * NSUB
STRIPE = 640
ZROWS = 32
TRASH = 10000
ROWS = 640


def _tc_layer1(x, m2, W1):
    def body(x_ref, m_ref, w_ref, o_ref):
        v = pl.program_id(0)
        xb = x_ref[...]
        mb = jnp.where(v == 0, 1.0, m_ref[0])
        res = jnp.dot(xb * mb, w_ref[...], preferred_element_type=jnp.float32)
        o_ref[0, 0] = res[:, :HALF]
        o_ref[1, 0] = res[:, HALF:]

    return pl.pallas_call(
        body,
        grid=(3, NP // ROWS),
        in_specs=[
            pl.BlockSpec((ROWS, D), lambda v, i: (i, 0)),
            pl.BlockSpec((1, ROWS, D), lambda v, i: (jnp.maximum(v - 1, 0), i, 0)),
            pl.BlockSpec((D, D), lambda v, i: (0, 0)),
        ],
        out_specs=pl.BlockSpec((2, 1, ROWS, HALF), lambda v, i: (0, v, i, 0)),
        out_shape=jax.ShapeDtypeStruct((2, 3, NP, HALF), jnp.float32),
    )(x, m2, W1)


def _tc_layer2(a1, W2):
    def body(lo_ref, hi_ref, w_ref, o_ref):
        lo = jnp.maximum(lo_ref[0, 0], 0.0)
        hi = jnp.maximum(hi_ref[0, 0], 0.0)
        w = w_ref[...]
        res = jnp.dot(lo, w[:HALF, :], preferred_element_type=jnp.float32)
        res += jnp.dot(hi, w[HALF:, :], preferred_element_type=jnp.float32)
        o_ref[0, 0] = res[:, :HALF]
        o_ref[1, 0] = res[:, HALF:]

    return pl.pallas_call(
        body,
        grid=(3, NP // ROWS),
        in_specs=[
            pl.BlockSpec((1, 1, ROWS, HALF), lambda v, i: (0, v, i, 0)),
            pl.BlockSpec((1, 1, ROWS, HALF), lambda v, i: (1, v, i, 0)),
            pl.BlockSpec((D, D), lambda v, i: (0, 0)),
        ],
        out_specs=pl.BlockSpec((2, 1, ROWS, HALF), lambda v, i: (0, v, i, 0)),
        out_shape=jax.ShapeDtypeStruct((2, 3, NP, HALF), jnp.float32),
    )(a1, a1, W2)


def _sc_aggregate(h_flat, src_t, dst_t, keep_t, bias2):
    mesh = plsc.VectorSubcoreMesh(core_axis_name="c", subcore_axis_name="s")

    @functools.partial(
        pl.kernel,
        mesh=mesh,
        out_type=jax.ShapeDtypeStruct((2, 3, NP, HALF), jnp.float32),
        scratch_types=[
            pltpu.VMEM((GRP, CH), jnp.int32),
            pltpu.VMEM((GRP, CH), jnp.int32),
            pltpu.VMEM((GRP, CH), jnp.int32),
            pltpu.VMEM((GRP, CH), jnp.int32),
            pltpu.VMEM((GRP, CH), jnp.int32),
            pltpu.VMEM((2, CH, HALF), jnp.float32),
            pltpu.VMEM((ZROWS, HALF), jnp.float32),
            pltpu.VMEM((HALF,), jnp.float32),
            pltpu.VMEM_SHARED((NP, HALF), jnp.float32),
            pltpu.SemaphoreType.DMA,
            pltpu.SemaphoreType.DMA,
        ],
    )
    def k(h_hbm, src_hbm, dst_hbm, keep_hbm, bias_hbm, out_hbm,
          sstage, dstage, kstage, gidxg, deffg, rows, zbuf, bvm, acc,
          sem0, sem1):
        c = lax.axis_index("c")
        s = lax.axis_index("s")
        pltpu.sync_copy(bias_hbm.at[c], bvm)
        bregs = [bvm[pl.ds(j * 16, 16)] for j in range(8)]

        def fillrow(i, carry):
            for j in range(8):
                zbuf[i, pl.ds(j * 16, 16)] = bregs[j]
            return carry

        lax.fori_loop(0, ZROWS, fillrow, 0)

        base = s * STRIPE
        for v in range(3):
            goff = (c * 3 + v) * NP

            for t in range(STRIPE // ZROWS):
                pltpu.sync_copy(zbuf, acc.at[pl.ds(base + t * ZROWS, ZROWS)])

            plsc.subcore_barrier()

            def group(g, carry):
                pltpu.sync_copy(src_hbm.at[s, g], sstage)
                pltpu.sync_copy(dst_hbm.at[s, g], dstage)
                pltpu.sync_copy(keep_hbm.at[v, s, g], kstage)

                def cmp(r, cc):
                    for j in range(8):
                        sl = pl.ds(j * 16, 16)
                        gidxg[r, sl] = sstage[r, sl] + goff
                        deffg[r, sl] = jnp.where(kstage[r, sl] != 0,
                                                 dstage[r, sl], TRASH)
                    return cc

                lax.fori_loop(0, GRP, cmp, 0)

                sems = (sem0, sem1)
                pltpu.async_copy(h_hbm.at[gidxg.at[0]], rows.at[0], sem0)
                for ch in range(GRP):
                    b = ch % 2
                    pltpu.make_async_copy(h_hbm.at[gidxg.at[ch]],
                                          rows.at[b], sems[b]).wait()
                    if ch + 1 < GRP:
                        nb = (ch + 1) % 2
                        pltpu.async_copy(h_hbm.at[gidxg.at[ch + 1]],
                                         rows.at[nb], sems[nb])
                    pltpu.sync_copy(rows.at[b], acc.at[deffg.at[ch]],
                                    add=True)
                return carry

            lax.fori_loop(0, NGRP, group, 0)

            plsc.subcore_barrier()

            pltpu.sync_copy(acc.at[pl.ds(base, STRIPE)],
                            out_hbm.at[c, v, pl.ds(base, STRIPE)])

    return k(h_flat, src_t, dst_t, keep_t, bias2)


def kernel(x, edge_index, W1, b1, W2, b2):
    src = edge_index[0]
    dst = edge_index[1]

    base = jax.random.key(42)
    kf1, kf2, ke1, ke2 = jax.random.split(base, 4)
    m1 = jax.random.bernoulli(kf1, 0.6, x.shape).astype(jnp.float32)
    m2 = jax.random.bernoulli(kf2, 0.6, x.shape).astype(jnp.float32)
    ew1 = jax.random.bernoulli(ke1, 0.6, (E,))
    ew2 = jax.random.bernoulli(ke2, 0.6, (E,))

    pad = EPAD - E
    src_t = jnp.pad(src, (0, pad)).reshape(NSUB, NGRP, GRP, CH)
    dst_t = jnp.pad(dst, (0, pad)).reshape(NSUB, NGRP, GRP, CH)
    keep3 = jnp.stack([
        jnp.ones((E,), jnp.int32),
        ew1.astype(jnp.int32),
        ew2.astype(jnp.int32),
    ])
    keep_t = jnp.pad(keep3, ((0, 0), (0, pad))).reshape(3, NSUB, NGRP, GRP, CH)

    x_p = jnp.pad(x, ((0, NP - N), (0, 0)))
    masks = jnp.pad(jnp.stack([m1, m2]), ((0, 0), (0, NP - N), (0, 0)))
    h4 = _tc_layer1(x_p, masks, W1)
    a1 = _sc_aggregate(h4.reshape(6 * NP, HALF), src_t, dst_t, keep_t,
                       b1.reshape(2, HALF))
    h24 = _tc_layer2(a1, W2)
    a2 = _sc_aggregate(h24.reshape(6 * NP, HALF), src_t, dst_t, keep_t,
                       b2.reshape(2, HALF))

    z = jnp.concatenate([a2[0, 0, :N], a2[1, 0, :N]], axis=1)
    z1 = jnp.concatenate([a2[0, 1, :N], a2[1, 1, :N]], axis=1)
    z2 = jnp.concatenate([a2[0, 2, :N], a2[1, 2, :N]], axis=1)
    return (z, z1, z2)


# speedup vs baseline: 2.0982x; 1.0271x over previous
import functools

import jax
import jax.numpy as jnp
from jax import lax
from jax.experimental import pallas as pl
from jax.experimental.pallas import tpu as pltpu
from jax.experimental.pallas import tpu_sc as plsc

N = 10000
NP = 10240
D = 256
E = 160000
HALF = 128
NSUB = 16
CH = 128
GRP = 8
NGRP = 10
EPT = CH * GRP * NGRP
EPAD = EPT * NSUB
STRIPE = 640
ZROWS = 32
TRASH = 10000
ROWS = 640


def _tc_layer1(x, m2, W1):
    def body(x_ref, m_ref, w_ref, o_ref):
        v = pl.program_id(0)
        xb = x_ref[...]
        mb = jnp.where(v == 0, 1.0, m_ref[0])
        res = jnp.dot(xb * mb, w_ref[...], preferred_element_type=jnp.float32)
        o_ref[0, 0] = res[:, :HALF]
        o_ref[1, 0] = res[:, HALF:]

    return pl.pallas_call(
        body,
        grid=(3, NP // ROWS),
        in_specs=[
            pl.BlockSpec((ROWS, D), lambda v, i: (i, 0)),
            pl.BlockSpec((1, ROWS, D), lambda v, i: (jnp.maximum(v - 1, 0), i, 0)),
            pl.BlockSpec((D, D), lambda v, i: (0, 0)),
        ],
        out_specs=pl.BlockSpec((2, 1, ROWS, HALF), lambda v, i: (0, v, i, 0)),
        out_shape=jax.ShapeDtypeStruct((2, 3, NP, HALF), jnp.float32),
    )(x, m2, W1)


def _tc_layer2(a1, W2):
    def body(lo_ref, hi_ref, w_ref, o_ref):
        lo = jnp.maximum(lo_ref[0, 0], 0.0)
        hi = jnp.maximum(hi_ref[0, 0], 0.0)
        w = w_ref[...]
        res = jnp.dot(lo, w[:HALF, :], preferred_element_type=jnp.float32)
        res += jnp.dot(hi, w[HALF:, :], preferred_element_type=jnp.float32)
        o_ref[0, 0] = res[:, :HALF]
        o_ref[1, 0] = res[:, HALF:]

    return pl.pallas_call(
        body,
        grid=(3, NP // ROWS),
        in_specs=[
            pl.BlockSpec((1, 1, ROWS, HALF), lambda v, i: (0, v, i, 0)),
            pl.BlockSpec((1, 1, ROWS, HALF), lambda v, i: (1, v, i, 0)),
            pl.BlockSpec((D, D), lambda v, i: (0, 0)),
        ],
        out_specs=pl.BlockSpec((2, 1, ROWS, HALF), lambda v, i: (0, v, i, 0)),
        out_shape=jax.ShapeDtypeStruct((2, 3, NP, HALF), jnp.float32),
    )(a1, a1, W2)


def _sc_aggregate(h_flat, src_t, dst_t, keep_t, bias2):
    mesh = plsc.VectorSubcoreMesh(core_axis_name="c", subcore_axis_name="s")

    @functools.partial(
        pl.kernel,
        mesh=mesh,
        out_type=jax.ShapeDtypeStruct((2, 3, NP, HALF), jnp.float32),
        scratch_types=[
            pltpu.VMEM((GRP, CH), jnp.int32),
            pltpu.VMEM((GRP, CH), jnp.int32),
            pltpu.VMEM((GRP, CH), jnp.int32),
            pltpu.VMEM((GRP, CH), jnp.int32),
            pltpu.VMEM((GRP, CH), jnp.int32),
            pltpu.VMEM((2, CH, HALF), jnp.float32),
            pltpu.VMEM((ZROWS, HALF), jnp.float32),
            pltpu.VMEM((HALF,), jnp.float32),
            pltpu.VMEM_SHARED((NP, HALF), jnp.float32),
            pltpu.SemaphoreType.DMA,
            pltpu.SemaphoreType.DMA,
        ],
    )
    def k(h_hbm, src_hbm, dst_hbm, keep_hbm, bias_hbm, out_hbm,
          sstage, dstage, kstage, gidxg, deffg, rows, zbuf, bvm, acc,
          sem0, sem1):
        c = lax.axis_index("c")
        s = lax.axis_index("s")
        pltpu.sync_copy(bias_hbm.at[c], bvm)
        bregs = [bvm[pl.ds(j * 16, 16)] for j in range(8)]

        def fillrow(i, carry):
            for j in range(8):
                zbuf[i, pl.ds(j * 16, 16)] = bregs[j]
            return carry

        lax.fori_loop(0, ZROWS, fillrow, 0)

        base = s * STRIPE
        for v in range(3):
            goff = (c * 3 + v) * NP

            for t in range(STRIPE // ZROWS):
                pltpu.sync_copy(zbuf, acc.at[pl.ds(base + t * ZROWS, ZROWS)])

            plsc.subcore_barrier()

            def group(g, carry):
                pltpu.sync_copy(src_hbm.at[s, g], sstage)
                pltpu.sync_copy(dst_hbm.at[s, g], dstage)
                pltpu.sync_copy(keep_hbm.at[v, s, g], kstage)

                def cmp(r, cc):
                    for j in range(8):
                        sl = pl.ds(j * 16, 16)
                        gidxg[r, sl] = sstage[r, sl] + goff
                        deffg[r, sl] = jnp.where(kstage[r, sl] != 0,
                                                 dstage[r, sl], TRASH)
                    return cc

                lax.fori_loop(0, GRP, cmp, 0)

                sems = (sem0, sem1)
                pltpu.async_copy(h_hbm.at[gidxg.at[0]], rows.at[0], sem0)
                pltpu.async_copy(h_hbm.at[gidxg.at[1]], rows.at[1], sem1)
                for ch in range(GRP):
                    b = ch % 2
                    pltpu.make_async_copy(h_hbm.at[gidxg.at[ch]],
                                          rows.at[b], sems[b]).wait()
                    pltpu.sync_copy(rows.at[b], acc.at[deffg.at[ch]],
                                    add=True)
                    if ch + 2 < GRP:
                        pltpu.async_copy(h_hbm.at[gidxg.at[ch + 2]],
                                         rows.at[b], sems[b])
                return carry

            lax.fori_loop(0, NGRP, group, 0)

            plsc.subcore_barrier()

            pltpu.sync_copy(acc.at[pl.ds(base, STRIPE)],
                            out_hbm.at[c, v, pl.ds(base, STRIPE)])

    return k(h_flat, src_t, dst_t, keep_t, bias2)


def kernel(x, edge_index, W1, b1, W2, b2):
    src = edge_index[0]
    dst = edge_index[1]

    base = jax.random.key(42)
    kf1, kf2, ke1, ke2 = jax.random.split(base, 4)
    m1 = jax.random.bernoulli(kf1, 0.6, x.shape).astype(jnp.float32)
    m2 = jax.random.bernoulli(kf2, 0.6, x.shape).astype(jnp.float32)
    ew1 = jax.random.bernoulli(ke1, 0.6, (E,))
    ew2 = jax.random.bernoulli(ke2, 0.6, (E,))

    pad = EPAD - E
    src_t = jnp.pad(src, (0, pad)).reshape(NSUB, NGRP, GRP, CH)
    dst_t = jnp.pad(dst, (0, pad)).reshape(NSUB, NGRP, GRP, CH)
    keep3 = jnp.stack([
        jnp.ones((E,), jnp.int32),
        ew1.astype(jnp.int32),
        ew2.astype(jnp.int32),
    ])
    keep_t = jnp.pad(keep3, ((0, 0), (0, pad))).reshape(3, NSUB, NGRP, GRP, CH)

    x_p = jnp.pad(x, ((0, NP - N), (0, 0)))
    masks = jnp.pad(jnp.stack([m1, m2]), ((0, 0), (0, NP - N), (0, 0)))
    h4 = _tc_layer1(x_p, masks, W1)
    a1 = _sc_aggregate(h4.reshape(6 * NP, HALF), src_t, dst_t, keep_t,
                       b1.reshape(2, HALF))
    h24 = _tc_layer2(a1, W2)
    a2 = _sc_aggregate(h24.reshape(6 * NP, HALF), src_t, dst_t, keep_t,
                       b2.reshape(2, HALF))

    z = jnp.concatenate([a2[0, 0, :N], a2[1, 0, :N]], axis=1)
    z1 = jnp.concatenate([a2[0, 1, :N], a2[1, 1, :N]], axis=1)
    z2 = jnp.concatenate([a2[0, 2, :N], a2[1, 2, :N]], axis=1)
    return (z, z1, z2)
